# Initial kernel scaffold; baseline (speedup 1.0000x reference)
import jax
import jax.numpy as jnp
from jax.experimental import pallas as pl


def kernel(x, Wq, bq, Wk, bk, Wv, bv, Wo, bo):
    raise NotImplementedError("write your pallas kernel here")


# speedup vs baseline: 1.6863x; 1.6863x over previous
import functools

import jax
import jax.numpy as jnp
from jax.experimental import pallas as pl

N_EMBD = 1024
N_HEAD = 16
HEAD_DIM = N_EMBD // N_HEAD
BLOCK = 64
SEQ = 2048
NB = SEQ // BLOCK


def _mm_bias_kernel(x_ref, w_ref, b_ref, o_ref):
    o_ref[...] = (
        jnp.dot(x_ref[...], w_ref[...], preferred_element_type=jnp.float32)
        + b_ref[...]
    )


def _mm_bias(x, w, b, tm, tn):
    m, k = x.shape
    k2, n = w.shape
    grid = (m // tm, n // tn)
    return pl.pallas_call(
        _mm_bias_kernel,
        grid=grid,
        in_specs=[
            pl.BlockSpec((tm, k), lambda i, j: (i, 0)),
            pl.BlockSpec((k, tn), lambda i, j: (0, j)),
            pl.BlockSpec((1, tn), lambda i, j: (0, j)),
        ],
        out_specs=pl.BlockSpec((tm, tn), lambda i, j: (i, j)),
        out_shape=jax.ShapeDtypeStruct((m, n), jnp.float32),
    )(x, w, b.reshape(1, -1))


def _attn_kernel(q_ref, k_ref, v_ref, o_ref, *, tq):
    t = pl.program_id(1)
    q = q_ref[0]
    k = k_ref[0]
    v = v_ref[0]

    s = jax.lax.dot_general(
        q, k, (((1,), (1,)), ((), ())), preferred_element_type=jnp.float32
    )
    s3 = s.reshape(tq, NB, BLOCK)
    m = jnp.max(s3, axis=-1, keepdims=True)
    e = jnp.exp(s3 - m)
    denom = jnp.sum(e, axis=-1, keepdims=True)
    p3 = e / denom

    row = jax.lax.broadcasted_iota(jnp.int32, (tq, NB, 1), 0)
    qblk = t * (tq // BLOCK) + row // BLOCK
    col = jax.lax.broadcasted_iota(jnp.int32, (tq, NB, 1), 1)
    p3 = jnp.where(col <= qblk, p3, 0.0)

    p = p3.reshape(tq, SEQ)
    o_ref[0] = jnp.dot(p, v, preferred_element_type=jnp.float32)


def _attention(q, k, v, tq):
    grid = (N_HEAD, SEQ // tq)
    return pl.pallas_call(
        functools.partial(_attn_kernel, tq=tq),
        grid=grid,
        in_specs=[
            pl.BlockSpec((1, tq, HEAD_DIM), lambda h, t: (h, t, 0)),
            pl.BlockSpec((1, SEQ, HEAD_DIM), lambda h, t: (h, 0, 0)),
            pl.BlockSpec((1, SEQ, HEAD_DIM), lambda h, t: (h, 0, 0)),
        ],
        out_specs=pl.BlockSpec((1, tq, HEAD_DIM), lambda h, t: (h, t, 0)),
        out_shape=jax.ShapeDtypeStruct((N_HEAD, SEQ, HEAD_DIM), jnp.float32),
    )(q, k, v)


def kernel(x, Wq, bq, Wk, bk, Wv, bv, Wo, bo):
    B, S, E = x.shape
    x2 = x.reshape(S, E)

    Wqkv = jnp.concatenate([Wq.T, Wk.T, Wv.T], axis=1)
    bqkv = jnp.concatenate([bq, bk, bv])

    qkv = _mm_bias(x2, Wqkv, bqkv, tm=256, tn=512)
    q, k, v = jnp.split(qkv, 3, axis=1)
    scale = 1.0 / (HEAD_DIM ** 0.5)
    q = (q * scale).reshape(S, N_HEAD, HEAD_DIM).transpose(1, 0, 2)
    k = k.reshape(S, N_HEAD, HEAD_DIM).transpose(1, 0, 2)
    v = v.reshape(S, N_HEAD, HEAD_DIM).transpose(1, 0, 2)

    o = _attention(q, k, v, tq=256)
    y = o.transpose(1, 0, 2).reshape(S, E)

    out = _mm_bias(y, Wo.T, bo, tm=256, tn=512)
    return out.reshape(B, S, E)
